# baseline (speedup 1.0000x reference)
import math
import functools

import jax
import jax.numpy as jnp
from jax import lax
from jax.experimental import pallas as pl
from jax.experimental.pallas import tpu as pltpu
from jax.experimental.pallas import tpu_sc as plsc

_NC = 2
_NS = 16
_NW = _NC * _NS
_L = 16

_EBLK = 128


def _prep_body(h_ref, wa_ref, wb_ref, be1_ref, A_ref, B_ref):
    hb = h_ref[...]
    A_ref[...] = (
        jnp.dot(hb, wa_ref[...], preferred_element_type=jnp.float32) + be1_ref[...]
    )
    B_ref[...] = jnp.dot(hb, wb_ref[...], preferred_element_type=jnp.float32)


def _prep(h, We1a, We1b, be1):
    N, D = h.shape
    BN = 2000
    grid = (N // BN,)
    return pl.pallas_call(
        _prep_body,
        grid=grid,
        in_specs=[
            pl.BlockSpec((BN, D), lambda i: (i, 0)),
            pl.BlockSpec((D, D), lambda i: (0, 0)),
            pl.BlockSpec((D, D), lambda i: (0, 0)),
            pl.BlockSpec((1, D), lambda i: (0, 0)),
        ],
        out_specs=[
            pl.BlockSpec((BN, D), lambda i: (i, 0)),
            pl.BlockSpec((BN, D), lambda i: (i, 0)),
        ],
        out_shape=[
            jax.ShapeDtypeStruct((N, D), jnp.float32),
            jax.ShapeDtypeStruct((N, D), jnp.float32),
        ],
    )(h, We1a, We1b, be1)


_NBUF = 3
_GBLK = 64


def _gather2(idxRC, Ap, Bp, E, D, N):
    nblk = E // _GBLK
    base = nblk // _NW
    extra = nblk % _NW
    assert base % _NBUF == 0
    DP = D // 2
    nrow_blk = N // 128
    nrow_tail = N - nrow_blk * 128
    st_base = nrow_blk // _NS
    st_extra = nrow_blk % _NS

    mesh = plsc.VectorSubcoreMesh(core_axis_name="c", subcore_axis_name="s")

    @functools.partial(
        pl.kernel,
        out_type=[
            jax.ShapeDtypeStruct((E, DP), jnp.int32),
            jax.ShapeDtypeStruct((E, DP), jnp.int32),
        ],
        mesh=mesh,
        scratch_types=[
            pltpu.VMEM_SHARED((N, DP), jnp.int32),
            pltpu.VMEM_SHARED((N, DP), jnp.int32),
            pltpu.VMEM((_NBUF, 2, _GBLK), jnp.int32),
            pltpu.VMEM((_NBUF, _GBLK, DP), jnp.int32),
            pltpu.VMEM((_NBUF, _GBLK, DP), jnp.int32),
        ] + [pltpu.SemaphoreType.DMA] * (3 * _NBUF),
    )
    def k(idx_hbm, A_hbm, B_hbm, G_hbm, H_hbm, shA, shB, ibuf, bufA, bufB,
          *sems):
        isem = sems[:_NBUF]
        gsem = sems[_NBUF:2 * _NBUF]
        wsem = sems[2 * _NBUF:]
        cid = lax.axis_index("c")
        sid = lax.axis_index("s")
        wid = sid * _NC + cid
        has_extra = wid < extra
        blk0 = wid * base + jnp.minimum(wid, extra)

        nst = st_base + jnp.where(sid < st_extra, 1, 0)

        def stage(j, _):
            blk = j * _NS + sid
            sl = pl.ds(blk * 128, 128)
            pltpu.sync_copy(A_hbm.at[sl], shA.at[sl])
            pltpu.sync_copy(B_hbm.at[sl], shB.at[sl])
            return 0

        lax.fori_loop(0, nst, stage, 0)
        if nrow_tail:
            @pl.when(sid == _NS - 1)
            def _():
                tsl = pl.ds(nrow_blk * 128, nrow_tail)
                pltpu.sync_copy(A_hbm.at[tsl], shA.at[tsl])
                pltpu.sync_copy(B_hbm.at[tsl], shB.at[tsl])
        plsc.subcore_barrier()

        def s_idx(j, p):
            pltpu.async_copy(idx_hbm.at[blk0 + j], ibuf.at[p], isem[p])

        def s_gather(j, p):
            pltpu.make_async_copy(idx_hbm.at[blk0 + j], ibuf.at[p], isem[p]).wait()
            pltpu.async_copy(shA.at[ibuf.at[p, 0]], bufA.at[p], gsem[p])
            pltpu.async_copy(shB.at[ibuf.at[p, 1]], bufB.at[p], gsem[p])

        def s_write(j, p):
            pltpu.make_async_copy(shA.at[ibuf.at[p, 0]], bufA.at[p], gsem[p]).wait()
            pltpu.make_async_copy(shB.at[ibuf.at[p, 1]], bufB.at[p], gsem[p]).wait()
            sl = pl.ds((blk0 + j) * _GBLK, _GBLK)
            pltpu.async_copy(bufA.at[p], G_hbm.at[sl], wsem[p])
            pltpu.async_copy(bufB.at[p], H_hbm.at[sl], wsem[p])

        def w_write(j, p):
            sl = pl.ds((blk0 + j) * _GBLK, _GBLK)
            pltpu.make_async_copy(bufA.at[p], G_hbm.at[sl], wsem[p]).wait()
            pltpu.make_async_copy(bufB.at[p], H_hbm.at[sl], wsem[p]).wait()

        s_idx(0, 0)
        s_idx(1, 1)
        s_gather(0, 0)
        s_idx(2, 2)
        s_gather(1, 1)
        s_write(0, 0)

        def group(g, _):
            j0 = g * _NBUF
            for o in range(_NBUF):
                j = j0 + o
                w_write(j - _NBUF, o)
                s_idx(j, o)
                s_gather(j - 1, (o - 1) % _NBUF)
                s_write(j - 2, (o - 2) % _NBUF)
            return 0

        lax.fori_loop(1, base // _NBUF, group, 0)
        w_write(base - _NBUF, 0)
        s_gather(base - 1, (base - 1) % _NBUF)
        s_write(base - 2, (base - 2) % _NBUF)
        w_write(base - 2, (base - 2) % _NBUF)
        s_write(base - 1, (base - 1) % _NBUF)
        w_write(base - 1, (base - 1) % _NBUF)
        if extra:
            @pl.when(has_extra)
            def _():
                s_idx(base, 1)
                s_gather(base, 1)
                s_write(base, 1)
                w_write(base, 1)

    return k(idxRC, Ap, Bp)


def _edge_body(sc_ref, g_ref, hh_ref, d_ref, em_ref, w256_ref, We2_ref,
               be2_ref, Wa_ref, ba_ref, ef_ref):
    left = sc_ref[0, 0]
    inv2 = sc_ref[0, 1]
    d = d_ref[...]
    em = em_ref[...]
    g = left * jnp.exp(-(d * d) * inv2) * em
    mask = jnp.int32(-65536)
    a32 = g_ref[...]
    b32 = hh_ref[...]
    xe = lax.bitcast_convert_type(jnp.left_shift(a32, 16), jnp.float32) + \
        lax.bitcast_convert_type(jnp.left_shift(b32, 16), jnp.float32)
    xo = lax.bitcast_convert_type(jnp.bitwise_and(a32, mask), jnp.float32) + \
        lax.bitcast_convert_type(jnp.bitwise_and(b32, mask), jnp.float32)
    x = jnp.concatenate([xe, xo], axis=1) + g * w256_ref[...]
    h1 = x * jax.nn.sigmoid(x)
    mij = jnp.dot(h1, We2_ref[...], preferred_element_type=jnp.float32) + be2_ref[...]
    att = jax.nn.sigmoid(
        jnp.dot(mij, Wa_ref[...], preferred_element_type=jnp.float32) + ba_ref[...]
    )
    ef_ref[...] = mij * att * em


def _edge_mlp(scalars, G32, H32, distances, edge_mask, w256p, We2p, be2, Wa, ba):
    E, DP = G32.shape
    D = 2 * DP
    BE = 3200
    grid = (E // BE,)
    return pl.pallas_call(
        _edge_body,
        grid=grid,
        in_specs=[
            pl.BlockSpec(memory_space=pltpu.SMEM),
            pl.BlockSpec((BE, DP), lambda i: (i, 0)),
            pl.BlockSpec((BE, DP), lambda i: (i, 0)),
            pl.BlockSpec((BE, 1), lambda i: (i, 0)),
            pl.BlockSpec((BE, 1), lambda i: (i, 0)),
            pl.BlockSpec((1, D), lambda i: (0, 0)),
            pl.BlockSpec((D, D), lambda i: (0, 0)),
            pl.BlockSpec((1, D), lambda i: (0, 0)),
            pl.BlockSpec((D, 1), lambda i: (0, 0)),
            pl.BlockSpec((1, 1), lambda i: (0, 0)),
        ],
        out_specs=pl.BlockSpec((BE, D), lambda i: (i, 0)),
        out_shape=jax.ShapeDtypeStruct((E, D), jnp.float32),
    )(scalars, G32, H32, distances, edge_mask, w256p, We2p, be2, Wa, ba)


def _scatter_add(ef, row3d, N, D):
    E = ef.shape[0]
    nblk = E // _EBLK
    base = nblk // _NW
    extra = nblk % _NW
    maxblk = base + (1 if extra else 0)
    nrow_blk = N // 128
    nrow_tail = N - nrow_blk * 128
    zb_base = nrow_blk // _NS
    zb_extra = nrow_blk % _NS

    mesh = plsc.VectorSubcoreMesh(core_axis_name="c", subcore_axis_name="s")

    @functools.partial(
        pl.kernel,
        out_type=[
            jax.ShapeDtypeStruct((N, D), jnp.float32),
            jax.ShapeDtypeStruct((N, D), jnp.float32),
        ],
        mesh=mesh,
        scratch_types=[
            pltpu.VMEM((maxblk, 1, _EBLK), jnp.int32),
            pltpu.VMEM((2, _EBLK, D), jnp.float32),
            pltpu.VMEM_SHARED((N, D), jnp.float32),
            pltpu.SemaphoreType.DMA,
            pltpu.SemaphoreType.DMA,
        ],
    )
    def k(ef_hbm, row_hbm, p0_hbm, p1_hbm, ridx, buf, agg, l0, l1):
        cid = lax.axis_index("c")
        sid = lax.axis_index("s")
        wid = sid * _NC + cid
        has_extra = wid < extra
        blk0 = wid * base + jnp.minimum(wid, extra)
        lsem = (l0, l1)

        def zrow(r, _):
            for c in range(D // _L):
                buf[0, r, pl.ds(c * _L, _L)] = jnp.zeros((_L,), jnp.float32)
            return 0

        lax.fori_loop(0, 128, zrow, 0)

        nz = zb_base + jnp.where(sid < zb_extra, 1, 0)

        def zcopy(j, _):
            blk = j * _NS + sid
            pltpu.sync_copy(buf.at[0], agg.at[pl.ds(blk * 128, 128)])
            return 0

        lax.fori_loop(0, nz, zcopy, 0)
        if nrow_tail:
            @pl.when(sid == _NS - 1)
            def _():
                pltpu.sync_copy(
                    buf.at[0].at[pl.ds(0, nrow_tail)],
                    agg.at[pl.ds(nrow_blk * 128, nrow_tail)],
                )
        plsc.subcore_barrier()

        pltpu.sync_copy(row_hbm.at[pl.ds(blk0, base)], ridx.at[pl.ds(0, base)])
        if extra:
            @pl.when(has_extra)
            def _():
                pltpu.sync_copy(
                    row_hbm.at[pl.ds(blk0 + base, 1)], ridx.at[pl.ds(base, 1)]
                )

        def start(j, p):
            pltpu.async_copy(
                ef_hbm.at[pl.ds((blk0 + j) * _EBLK, _EBLK)], buf.at[p], lsem[p]
            )

        def process(j, p):
            pltpu.make_async_copy(
                ef_hbm.at[pl.ds((blk0 + j) * _EBLK, _EBLK)], buf.at[p], lsem[p]
            ).wait()
            pltpu.sync_copy(buf.at[p], agg.at[ridx.at[j, 0]], add=True)

        start(0, 0)
        for j in range(1, base):
            start(j, j & 1)
            process(j - 1, (j - 1) & 1)
        if extra:
            @pl.when(has_extra)
            def _():
                start(base, base & 1)
        process(base - 1, (base - 1) & 1)
        if extra:
            @pl.when(has_extra)
            def _():
                process(base, base & 1)
        plsc.subcore_barrier()

        def wcopy(j, _):
            blk = j * _NS + sid
            sl = pl.ds(blk * 128, 128)

            @pl.when(cid == 0)
            def _():
                pltpu.sync_copy(agg.at[sl], p0_hbm.at[sl])

            @pl.when(cid == 1)
            def _():
                pltpu.sync_copy(agg.at[sl], p1_hbm.at[sl])

            return 0

        lax.fori_loop(0, nz, wcopy, 0)
        if nrow_tail:
            @pl.when(sid == _NS - 1)
            def _():
                tsl = pl.ds(nrow_blk * 128, nrow_tail)

                @pl.when(cid == 0)
                def _():
                    pltpu.sync_copy(agg.at[tsl], p0_hbm.at[tsl])

                @pl.when(cid == 1)
                def _():
                    pltpu.sync_copy(agg.at[tsl], p1_hbm.at[tsl])

    return k(ef, row3d)


def _node_body(h_ref, p0_ref, p1_ref, wna_ref, wnb_ref, bn1_ref, Wn2_ref,
               bn2_ref, Wl_ref, bl_ref, out_ref):
    hb = h_ref[...]
    agg = (p0_ref[...] + p1_ref[...]) * 0.01
    t = (
        jnp.dot(hb, wna_ref[...], preferred_element_type=jnp.float32)
        + jnp.dot(agg, wnb_ref[...], preferred_element_type=jnp.float32)
        + bn1_ref[...]
    )
    t = t * jax.nn.sigmoid(t)
    out = hb + jnp.dot(t, Wn2_ref[...], preferred_element_type=jnp.float32) + bn2_ref[...]
    out_ref[...] = (
        jnp.dot(out, Wl_ref[...], preferred_element_type=jnp.float32) + bl_ref[...]
    )


def _node_mlp(h, p0, p1, Wn1a, Wn1b, bn1, Wn2, bn2, Wl, bl):
    N, D = h.shape
    BN = 2000
    grid = (N // BN,)
    bspec_nd = pl.BlockSpec((BN, D), lambda i: (i, 0))
    bspec_w = pl.BlockSpec((D, D), lambda i: (0, 0))
    bspec_b = pl.BlockSpec((1, D), lambda i: (0, 0))
    return pl.pallas_call(
        _node_body,
        grid=grid,
        in_specs=[
            bspec_nd, bspec_nd, bspec_nd,
            bspec_w, bspec_w, bspec_b,
            bspec_w, bspec_b, bspec_w, bspec_b,
        ],
        out_specs=bspec_nd,
        out_shape=jax.ShapeDtypeStruct((N, D), jnp.float32),
    )(h, p0, p1, Wn1a, Wn1b, bn1, Wn2, bn2, Wl, bl)


def kernel(h, distances, edges, node_mask, edge_mask, h_gauss, W_lin, b_lin,
           We1, be1, We2, be2, Wn1, bn1, Wn2, bn2, Wa, ba):
    N, D = h.shape
    E = distances.shape[0]

    row = edges[0].astype(jnp.int32)
    col = edges[1].astype(jnp.int32)
    row3d = row.reshape(E // _EBLK, 1, _EBLK)
    idxRC = jnp.stack(
        [row.reshape(E // _GBLK, _GBLK), col.reshape(E // _GBLK, _GBLK)], axis=1
    )

    hh = jax.nn.softplus(h_gauss)[0]
    left = 1.0 / (math.sqrt(2.0 * math.pi) * hh)
    inv2 = 1.0 / (2.0 * hh * hh)
    scalars = jnp.stack([left, inv2]).reshape(1, 2)

    We1a = We1[:D]
    We1b = We1[D:2 * D]
    w256 = We1[2 * D:]
    be1_r = be1.reshape(1, D)
    be2_r = be2.reshape(1, D)
    bn1_r = bn1.reshape(1, D)
    bn2_r = bn2.reshape(1, D)
    bl_r = b_lin.reshape(1, D)
    ba_r = ba.reshape(1, 1)
    Wn1a = Wn1[:D]
    Wn1b = Wn1[D:]

    A, B = _prep(h, We1a, We1b, be1_r)
    Ap = lax.bitcast_convert_type(
        A.astype(jnp.bfloat16).reshape(N, D // 2, 2), jnp.int32
    )
    Bp = lax.bitcast_convert_type(
        B.astype(jnp.bfloat16).reshape(N, D // 2, 2), jnp.int32
    )
    perm = jnp.concatenate(
        [jnp.arange(0, D, 2, dtype=jnp.int32), jnp.arange(1, D, 2, dtype=jnp.int32)]
    )
    w256p = w256[:, perm]
    We2p = We2[perm, :]

    G32, H32 = _gather2(idxRC, Ap, Bp, E, D, N)
    ef = _edge_mlp(scalars, G32, H32, distances, edge_mask, w256p, We2p, be2_r,
                   Wa, ba_r)
    p0, p1 = _scatter_add(ef, row3d, N, D)
    hidden = _node_mlp(h, p0, p1, Wn1a, Wn1b, bn1_r, Wn2, bn2_r, W_lin, bl_r)

    return (hidden, distances, edges, node_mask, edge_mask)


# speedup vs baseline: 1.0025x; 1.0025x over previous
import math
import functools

import jax
import jax.numpy as jnp
from jax import lax
from jax.experimental import pallas as pl
from jax.experimental.pallas import tpu as pltpu
from jax.experimental.pallas import tpu_sc as plsc

_NC = 2
_NS = 16
_NW = _NC * _NS
_L = 16

_EBLK = 128


def _prep_body(h_ref, wa_ref, wb_ref, be1_ref, A_ref, B_ref):
    hb = h_ref[...]
    A_ref[...] = (
        jnp.dot(hb, wa_ref[...], preferred_element_type=jnp.float32) + be1_ref[...]
    )
    B_ref[...] = jnp.dot(hb, wb_ref[...], preferred_element_type=jnp.float32)


def _prep(h, We1a, We1b, be1):
    N, D = h.shape
    BN = 2000
    grid = (N // BN,)
    return pl.pallas_call(
        _prep_body,
        grid=grid,
        in_specs=[
            pl.BlockSpec((BN, D), lambda i: (i, 0)),
            pl.BlockSpec((D, D), lambda i: (0, 0)),
            pl.BlockSpec((D, D), lambda i: (0, 0)),
            pl.BlockSpec((1, D), lambda i: (0, 0)),
        ],
        out_specs=[
            pl.BlockSpec((BN, D), lambda i: (i, 0)),
            pl.BlockSpec((BN, D), lambda i: (i, 0)),
        ],
        out_shape=[
            jax.ShapeDtypeStruct((N, D), jnp.float32),
            jax.ShapeDtypeStruct((N, D), jnp.float32),
        ],
    )(h, We1a, We1b, be1)


_NBUF = 3
_GBLK = 32


def _gather2(idxRC, Ap, Bp, E, D, N):
    nblk = E // _GBLK
    base = nblk // _NW
    extra = nblk % _NW
    assert base % _NBUF == 0
    DP = D // 2
    nrow_blk = N // 128
    nrow_tail = N - nrow_blk * 128
    st_base = nrow_blk // _NS
    st_extra = nrow_blk % _NS

    mesh = plsc.VectorSubcoreMesh(core_axis_name="c", subcore_axis_name="s")

    @functools.partial(
        pl.kernel,
        out_type=jax.ShapeDtypeStruct((E, D), jnp.int32),
        mesh=mesh,
        scratch_types=[
            pltpu.VMEM_SHARED((N, DP), jnp.int32),
            pltpu.VMEM_SHARED((N, DP), jnp.int32),
            pltpu.VMEM((_NBUF, 2, _GBLK), jnp.int32),
            pltpu.VMEM((_NBUF, _GBLK, DP), jnp.int32),
            pltpu.VMEM((_NBUF, _GBLK, DP), jnp.int32),
            pltpu.VMEM((_NBUF, _GBLK, D), jnp.int32),
        ] + [pltpu.SemaphoreType.DMA] * (3 * _NBUF),
    )
    def k(idx_hbm, A_hbm, B_hbm, X_hbm, shA, shB, ibuf, bufA, bufB, bufX,
          *sems):
        isem = sems[:_NBUF]
        gsem = sems[_NBUF:2 * _NBUF]
        wsem = sems[2 * _NBUF:]
        cid = lax.axis_index("c")
        sid = lax.axis_index("s")
        wid = sid * _NC + cid
        has_extra = wid < extra
        blk0 = wid * base + jnp.minimum(wid, extra)

        nst = st_base + jnp.where(sid < st_extra, 1, 0)

        def stage(j, _):
            blk = j * _NS + sid
            sl = pl.ds(blk * 128, 128)
            pltpu.sync_copy(A_hbm.at[sl], shA.at[sl])
            pltpu.sync_copy(B_hbm.at[sl], shB.at[sl])
            return 0

        lax.fori_loop(0, nst, stage, 0)
        if nrow_tail:
            @pl.when(sid == _NS - 1)
            def _():
                tsl = pl.ds(nrow_blk * 128, nrow_tail)
                pltpu.sync_copy(A_hbm.at[tsl], shA.at[tsl])
                pltpu.sync_copy(B_hbm.at[tsl], shB.at[tsl])
        plsc.subcore_barrier()

        def s_idx(j, p):
            pltpu.async_copy(idx_hbm.at[blk0 + j], ibuf.at[p], isem[p])

        def s_gather(j, p):
            pltpu.make_async_copy(idx_hbm.at[blk0 + j], ibuf.at[p], isem[p]).wait()
            pltpu.async_copy(shA.at[ibuf.at[p, 0]], bufA.at[p], gsem[p])
            pltpu.async_copy(shB.at[ibuf.at[p, 1]], bufB.at[p], gsem[p])

        def s_write(j, p):
            pltpu.make_async_copy(shA.at[ibuf.at[p, 0]], bufA.at[p], gsem[p]).wait()
            pltpu.make_async_copy(shB.at[ibuf.at[p, 1]], bufB.at[p], gsem[p]).wait()

            def row_body(r, _):
                for c in range(DP // _L):
                    sl = pl.ds(c * _L, _L)
                    bufX[p, r, sl] = bufA[p, r, sl]
                    bufX[p, r, pl.ds(DP + c * _L, _L)] = bufB[p, r, sl]
                return 0

            lax.fori_loop(0, _GBLK, row_body, 0)
            sl = pl.ds((blk0 + j) * _GBLK, _GBLK)
            pltpu.async_copy(bufX.at[p], X_hbm.at[sl], wsem[p])

        def w_write(j, p):
            sl = pl.ds((blk0 + j) * _GBLK, _GBLK)
            pltpu.make_async_copy(bufX.at[p], X_hbm.at[sl], wsem[p]).wait()

        s_idx(0, 0)
        s_idx(1, 1)
        s_gather(0, 0)
        s_idx(2, 2)
        s_gather(1, 1)
        s_write(0, 0)

        def group(g, _):
            j0 = g * _NBUF
            for o in range(_NBUF):
                j = j0 + o
                w_write(j - _NBUF, o)
                s_idx(j, o)
                s_gather(j - 1, (o - 1) % _NBUF)
                s_write(j - 2, (o - 2) % _NBUF)
            return 0

        lax.fori_loop(1, base // _NBUF, group, 0)
        w_write(base - _NBUF, 0)
        s_gather(base - 1, (base - 1) % _NBUF)
        s_write(base - 2, (base - 2) % _NBUF)
        w_write(base - 2, (base - 2) % _NBUF)
        s_write(base - 1, (base - 1) % _NBUF)
        w_write(base - 1, (base - 1) % _NBUF)
        if extra:
            @pl.when(has_extra)
            def _():
                s_idx(base, 1)
                s_gather(base, 1)
                s_write(base, 1)
                w_write(base, 1)

    return k(idxRC, Ap, Bp)


def _edge_body(sc_ref, x_ref, d_ref, em_ref, w256_ref, We2_ref,
               be2_ref, Wa_ref, ba_ref, ef_ref):
    left = sc_ref[0, 0]
    inv2 = sc_ref[0, 1]
    d = d_ref[...]
    em = em_ref[...]
    g = left * jnp.exp(-(d * d) * inv2) * em
    mask = jnp.int32(-65536)
    x32 = x_ref[...]
    DP = x32.shape[1] // 2
    a32 = x32[:, :DP]
    b32 = x32[:, DP:]
    xe = lax.bitcast_convert_type(jnp.left_shift(a32, 16), jnp.float32) + \
        lax.bitcast_convert_type(jnp.left_shift(b32, 16), jnp.float32)
    xo = lax.bitcast_convert_type(jnp.bitwise_and(a32, mask), jnp.float32) + \
        lax.bitcast_convert_type(jnp.bitwise_and(b32, mask), jnp.float32)
    x = jnp.concatenate([xe, xo], axis=1) + g * w256_ref[...]
    h1 = x * jax.nn.sigmoid(x)
    mij = jnp.dot(h1, We2_ref[...], preferred_element_type=jnp.float32) + be2_ref[...]
    att = jax.nn.sigmoid(
        jnp.dot(mij, Wa_ref[...], preferred_element_type=jnp.float32) + ba_ref[...]
    )
    ef_ref[...] = mij * att * em


def _edge_mlp(scalars, X32, distances, edge_mask, w256p, We2p, be2, Wa, ba):
    E, D = X32.shape
    BE = 3200
    grid = (E // BE,)
    return pl.pallas_call(
        _edge_body,
        grid=grid,
        in_specs=[
            pl.BlockSpec(memory_space=pltpu.SMEM),
            pl.BlockSpec((BE, D), lambda i: (i, 0)),
            pl.BlockSpec((BE, 1), lambda i: (i, 0)),
            pl.BlockSpec((BE, 1), lambda i: (i, 0)),
            pl.BlockSpec((1, D), lambda i: (0, 0)),
            pl.BlockSpec((D, D), lambda i: (0, 0)),
            pl.BlockSpec((1, D), lambda i: (0, 0)),
            pl.BlockSpec((D, 1), lambda i: (0, 0)),
            pl.BlockSpec((1, 1), lambda i: (0, 0)),
        ],
        out_specs=pl.BlockSpec((BE, D), lambda i: (i, 0)),
        out_shape=jax.ShapeDtypeStruct((E, D), jnp.float32),
    )(scalars, X32, distances, edge_mask, w256p, We2p, be2, Wa, ba)


def _scatter_add(ef, row3d, N, D):
    E = ef.shape[0]
    nblk = E // _EBLK
    base = nblk // _NW
    extra = nblk % _NW
    maxblk = base + (1 if extra else 0)
    nrow_blk = N // 128
    nrow_tail = N - nrow_blk * 128
    zb_base = nrow_blk // _NS
    zb_extra = nrow_blk % _NS

    mesh = plsc.VectorSubcoreMesh(core_axis_name="c", subcore_axis_name="s")

    @functools.partial(
        pl.kernel,
        out_type=[
            jax.ShapeDtypeStruct((N, D), jnp.float32),
            jax.ShapeDtypeStruct((N, D), jnp.float32),
        ],
        mesh=mesh,
        scratch_types=[
            pltpu.VMEM((maxblk, 1, _EBLK), jnp.int32),
            pltpu.VMEM((2, _EBLK, D), jnp.float32),
            pltpu.VMEM_SHARED((N, D), jnp.float32),
            pltpu.SemaphoreType.DMA,
            pltpu.SemaphoreType.DMA,
        ],
    )
    def k(ef_hbm, row_hbm, p0_hbm, p1_hbm, ridx, buf, agg, l0, l1):
        cid = lax.axis_index("c")
        sid = lax.axis_index("s")
        wid = sid * _NC + cid
        has_extra = wid < extra
        blk0 = wid * base + jnp.minimum(wid, extra)
        lsem = (l0, l1)

        def zrow(r, _):
            for c in range(D // _L):
                buf[0, r, pl.ds(c * _L, _L)] = jnp.zeros((_L,), jnp.float32)
            return 0

        lax.fori_loop(0, 128, zrow, 0)

        nz = zb_base + jnp.where(sid < zb_extra, 1, 0)

        def zcopy(j, _):
            blk = j * _NS + sid
            pltpu.sync_copy(buf.at[0], agg.at[pl.ds(blk * 128, 128)])
            return 0

        lax.fori_loop(0, nz, zcopy, 0)
        if nrow_tail:
            @pl.when(sid == _NS - 1)
            def _():
                pltpu.sync_copy(
                    buf.at[0].at[pl.ds(0, nrow_tail)],
                    agg.at[pl.ds(nrow_blk * 128, nrow_tail)],
                )
        plsc.subcore_barrier()

        pltpu.sync_copy(row_hbm.at[pl.ds(blk0, base)], ridx.at[pl.ds(0, base)])
        if extra:
            @pl.when(has_extra)
            def _():
                pltpu.sync_copy(
                    row_hbm.at[pl.ds(blk0 + base, 1)], ridx.at[pl.ds(base, 1)]
                )

        def start(j, p):
            pltpu.async_copy(
                ef_hbm.at[pl.ds((blk0 + j) * _EBLK, _EBLK)], buf.at[p], lsem[p]
            )

        def process(j, p):
            pltpu.make_async_copy(
                ef_hbm.at[pl.ds((blk0 + j) * _EBLK, _EBLK)], buf.at[p], lsem[p]
            ).wait()
            pltpu.sync_copy(buf.at[p], agg.at[ridx.at[j, 0]], add=True)

        start(0, 0)
        for j in range(1, base):
            start(j, j & 1)
            process(j - 1, (j - 1) & 1)
        if extra:
            @pl.when(has_extra)
            def _():
                start(base, base & 1)
        process(base - 1, (base - 1) & 1)
        if extra:
            @pl.when(has_extra)
            def _():
                process(base, base & 1)
        plsc.subcore_barrier()

        def wcopy(j, _):
            blk = j * _NS + sid
            sl = pl.ds(blk * 128, 128)

            @pl.when(cid == 0)
            def _():
                pltpu.sync_copy(agg.at[sl], p0_hbm.at[sl])

            @pl.when(cid == 1)
            def _():
                pltpu.sync_copy(agg.at[sl], p1_hbm.at[sl])

            return 0

        lax.fori_loop(0, nz, wcopy, 0)
        if nrow_tail:
            @pl.when(sid == _NS - 1)
            def _():
                tsl = pl.ds(nrow_blk * 128, nrow_tail)

                @pl.when(cid == 0)
                def _():
                    pltpu.sync_copy(agg.at[tsl], p0_hbm.at[tsl])

                @pl.when(cid == 1)
                def _():
                    pltpu.sync_copy(agg.at[tsl], p1_hbm.at[tsl])

    return k(ef, row3d)


def _node_body(h_ref, p0_ref, p1_ref, wna_ref, wnb_ref, bn1_ref, Wn2_ref,
               bn2_ref, Wl_ref, bl_ref, out_ref):
    hb = h_ref[...]
    agg = (p0_ref[...] + p1_ref[...]) * 0.01
    t = (
        jnp.dot(hb, wna_ref[...], preferred_element_type=jnp.float32)
        + jnp.dot(agg, wnb_ref[...], preferred_element_type=jnp.float32)
        + bn1_ref[...]
    )
    t = t * jax.nn.sigmoid(t)
    out = hb + jnp.dot(t, Wn2_ref[...], preferred_element_type=jnp.float32) + bn2_ref[...]
    out_ref[...] = (
        jnp.dot(out, Wl_ref[...], preferred_element_type=jnp.float32) + bl_ref[...]
    )


def _node_mlp(h, p0, p1, Wn1a, Wn1b, bn1, Wn2, bn2, Wl, bl):
    N, D = h.shape
    BN = 2000
    grid = (N // BN,)
    bspec_nd = pl.BlockSpec((BN, D), lambda i: (i, 0))
    bspec_w = pl.BlockSpec((D, D), lambda i: (0, 0))
    bspec_b = pl.BlockSpec((1, D), lambda i: (0, 0))
    return pl.pallas_call(
        _node_body,
        grid=grid,
        in_specs=[
            bspec_nd, bspec_nd, bspec_nd,
            bspec_w, bspec_w, bspec_b,
            bspec_w, bspec_b, bspec_w, bspec_b,
        ],
        out_specs=bspec_nd,
        out_shape=jax.ShapeDtypeStruct((N, D), jnp.float32),
    )(h, p0, p1, Wn1a, Wn1b, bn1, Wn2, bn2, Wl, bl)


def kernel(h, distances, edges, node_mask, edge_mask, h_gauss, W_lin, b_lin,
           We1, be1, We2, be2, Wn1, bn1, Wn2, bn2, Wa, ba):
    N, D = h.shape
    E = distances.shape[0]

    row = edges[0].astype(jnp.int32)
    col = edges[1].astype(jnp.int32)
    row3d = row.reshape(E // _EBLK, 1, _EBLK)
    idxRC = jnp.stack(
        [row.reshape(E // _GBLK, _GBLK), col.reshape(E // _GBLK, _GBLK)], axis=1
    )

    hh = jax.nn.softplus(h_gauss)[0]
    left = 1.0 / (math.sqrt(2.0 * math.pi) * hh)
    inv2 = 1.0 / (2.0 * hh * hh)
    scalars = jnp.stack([left, inv2]).reshape(1, 2)

    We1a = We1[:D]
    We1b = We1[D:2 * D]
    w256 = We1[2 * D:]
    be1_r = be1.reshape(1, D)
    be2_r = be2.reshape(1, D)
    bn1_r = bn1.reshape(1, D)
    bn2_r = bn2.reshape(1, D)
    bl_r = b_lin.reshape(1, D)
    ba_r = ba.reshape(1, 1)
    Wn1a = Wn1[:D]
    Wn1b = Wn1[D:]

    A, B = _prep(h, We1a, We1b, be1_r)
    Ap = lax.bitcast_convert_type(
        A.astype(jnp.bfloat16).reshape(N, D // 2, 2), jnp.int32
    )
    Bp = lax.bitcast_convert_type(
        B.astype(jnp.bfloat16).reshape(N, D // 2, 2), jnp.int32
    )
    perm = jnp.concatenate(
        [jnp.arange(0, D, 2, dtype=jnp.int32), jnp.arange(1, D, 2, dtype=jnp.int32)]
    )
    w256p = w256[:, perm]
    We2p = We2[perm, :]

    X32 = _gather2(idxRC, Ap, Bp, E, D, N)
    ef = _edge_mlp(scalars, X32, distances, edge_mask, w256p, We2p, be2_r,
                   Wa, ba_r)
    p0, p1 = _scatter_add(ef, row3d, N, D)
    hidden = _node_mlp(h, p0, p1, Wn1a, Wn1b, bn1_r, Wn2, bn2_r, W_lin, bl_r)

    return (hidden, distances, edges, node_mask, edge_mask)


# speedup vs baseline: 1.0778x; 1.0751x over previous
import math
import functools

import jax
import jax.numpy as jnp
from jax import lax
from jax.experimental import pallas as pl
from jax.experimental.pallas import tpu as pltpu
from jax.experimental.pallas import tpu_sc as plsc

_NC = 2
_NS = 16
_NW = _NC * _NS
_L = 16

_EBLK = 128


def _prep_body(h_ref, wa_ref, wb_ref, be1_ref, A_ref, B_ref):
    hb = h_ref[...]
    A_ref[...] = (
        jnp.dot(hb, wa_ref[...], preferred_element_type=jnp.float32) + be1_ref[...]
    )
    B_ref[...] = jnp.dot(hb, wb_ref[...], preferred_element_type=jnp.float32)


def _prep(h, We1a, We1b, be1):
    N, D = h.shape
    BN = 2000
    grid = (N // BN,)
    return pl.pallas_call(
        _prep_body,
        grid=grid,
        in_specs=[
            pl.BlockSpec((BN, D), lambda i: (i, 0)),
            pl.BlockSpec((D, D), lambda i: (0, 0)),
            pl.BlockSpec((D, D), lambda i: (0, 0)),
            pl.BlockSpec((1, D), lambda i: (0, 0)),
        ],
        out_specs=[
            pl.BlockSpec((BN, D), lambda i: (i, 0)),
            pl.BlockSpec((BN, D), lambda i: (i, 0)),
        ],
        out_shape=[
            jax.ShapeDtypeStruct((N, D), jnp.float32),
            jax.ShapeDtypeStruct((N, D), jnp.float32),
        ],
    )(h, We1a, We1b, be1)


def _gather_add(row3d, col3d, A, B, E, D):
    nblk = E // _EBLK
    base = nblk // _NW
    extra = nblk % _NW
    maxblk = base + (1 if extra else 0)

    mesh = plsc.VectorSubcoreMesh(core_axis_name="c", subcore_axis_name="s")

    @functools.partial(
        pl.kernel,
        out_type=jax.ShapeDtypeStruct((E, D), jnp.float32),
        mesh=mesh,
        scratch_types=[
            pltpu.VMEM((maxblk, 1, _EBLK), jnp.int32),
            pltpu.VMEM((maxblk, 1, _EBLK), jnp.int32),
            pltpu.VMEM((2, _EBLK, D), jnp.float32),
            pltpu.VMEM((2, _EBLK, D), jnp.float32),
            pltpu.SemaphoreType.DMA,
            pltpu.SemaphoreType.DMA,
            pltpu.SemaphoreType.DMA,
            pltpu.SemaphoreType.DMA,
        ],
    )
    def k(row_hbm, col_hbm, A_hbm, B_hbm, pre_hbm, ridx, cidx, bufA, bufB,
          g0, g1, w0, w1):
        wid = lax.axis_index("s") * _NC + lax.axis_index("c")
        has_extra = wid < extra
        blk0 = wid * base + jnp.minimum(wid, extra)

        gsem = (g0, g1)
        wsem = (w0, w1)

        pltpu.sync_copy(row_hbm.at[pl.ds(blk0, base)], ridx.at[pl.ds(0, base)])
        pltpu.sync_copy(col_hbm.at[pl.ds(blk0, base)], cidx.at[pl.ds(0, base)])
        if extra:
            @pl.when(has_extra)
            def _():
                pltpu.sync_copy(
                    row_hbm.at[pl.ds(blk0 + base, 1)], ridx.at[pl.ds(base, 1)]
                )
                pltpu.sync_copy(
                    col_hbm.at[pl.ds(blk0 + base, 1)], cidx.at[pl.ds(base, 1)]
                )

        def start(j, p):
            pltpu.async_copy(A_hbm.at[ridx.at[j, 0]], bufA.at[p], gsem[p])
            pltpu.async_copy(B_hbm.at[cidx.at[j, 0]], bufB.at[p], gsem[p])

        def process(j, p):
            pltpu.make_async_copy(A_hbm.at[ridx.at[j, 0]], bufA.at[p], gsem[p]).wait()
            pltpu.make_async_copy(B_hbm.at[cidx.at[j, 0]], bufB.at[p], gsem[p]).wait()

            def row_body(r, _):
                for c in range(D // _L):
                    sl = pl.ds(c * _L, _L)
                    bufA[p, r, sl] = bufA[p, r, sl] + bufB[p, r, sl]
                return 0

            lax.fori_loop(0, _EBLK, row_body, 0)
            pltpu.async_copy(
                bufA.at[p], pre_hbm.at[pl.ds((blk0 + j) * _EBLK, _EBLK)], wsem[p]
            )

        def wait_write(j, p):
            pltpu.make_async_copy(
                bufA.at[p], pre_hbm.at[pl.ds((blk0 + j) * _EBLK, _EBLK)], wsem[p]
            ).wait()

        start(0, 0)
        start(1, 1)
        process(0, 0)
        for j in range(2, base):
            p = j & 1
            wait_write(j - 2, p)
            start(j, p)
            process(j - 1, 1 - p)
        if extra:
            p = base & 1
            @pl.when(has_extra)
            def _():
                wait_write(base - 2, p)
                start(base, p)
        process(base - 1, (base - 1) & 1)
        if extra:
            @pl.when(has_extra)
            def _():
                process(base, base & 1)
        wait_write(base - 1, (base - 1) & 1)
        if extra:
            @pl.when(has_extra)
            def _():
                wait_write(base, base & 1)

            @pl.when(jnp.logical_not(has_extra))
            def _():
                wait_write(base - 2, base & 1)
        else:
            wait_write(base - 2, base & 1)

    return k(row3d, col3d, A, B)


def _edge_body(sc_ref, pre_ref, d_ref, em_ref, w256_ref, We2_ref,
               be2_ref, Wa_ref, ba_ref, ef_ref):
    left = sc_ref[0, 0]
    inv2 = sc_ref[0, 1]
    d = d_ref[...]
    em = em_ref[...]
    g = left * jnp.exp(-(d * d) * inv2) * em
    x = pre_ref[...] + g * w256_ref[...]
    h1 = x * jax.nn.sigmoid(x)
    mij = jnp.dot(
        h1.astype(jnp.bfloat16), We2_ref[...],
        preferred_element_type=jnp.float32,
    ) + be2_ref[...]
    att = jax.nn.sigmoid(
        jnp.dot(mij, Wa_ref[...], preferred_element_type=jnp.float32) + ba_ref[...]
    )
    ef_ref[...] = mij * att * em


def _edge_mlp(scalars, pre, distances, edge_mask, w256, We2, be2, Wa, ba):
    E, D = pre.shape
    BE = 3200
    grid = (E // BE,)
    return pl.pallas_call(
        _edge_body,
        grid=grid,
        in_specs=[
            pl.BlockSpec(memory_space=pltpu.SMEM),
            pl.BlockSpec((BE, D), lambda i: (i, 0)),
            pl.BlockSpec((BE, 1), lambda i: (i, 0)),
            pl.BlockSpec((BE, 1), lambda i: (i, 0)),
            pl.BlockSpec((1, D), lambda i: (0, 0)),
            pl.BlockSpec((D, D), lambda i: (0, 0)),
            pl.BlockSpec((1, D), lambda i: (0, 0)),
            pl.BlockSpec((D, 1), lambda i: (0, 0)),
            pl.BlockSpec((1, 1), lambda i: (0, 0)),
        ],
        out_specs=pl.BlockSpec((BE, D), lambda i: (i, 0)),
        out_shape=jax.ShapeDtypeStruct((E, D), jnp.float32),
    )(scalars, pre, distances, edge_mask, w256, We2, be2, Wa, ba)


def _scatter_add(ef, row3d, N, D):
    E = ef.shape[0]
    nblk = E // _EBLK
    base = nblk // _NW
    extra = nblk % _NW
    maxblk = base + (1 if extra else 0)
    nrow_blk = N // 128
    nrow_tail = N - nrow_blk * 128
    zb_base = nrow_blk // _NS
    zb_extra = nrow_blk % _NS

    mesh = plsc.VectorSubcoreMesh(core_axis_name="c", subcore_axis_name="s")

    @functools.partial(
        pl.kernel,
        out_type=[
            jax.ShapeDtypeStruct((N, D), jnp.float32),
            jax.ShapeDtypeStruct((N, D), jnp.float32),
        ],
        mesh=mesh,
        scratch_types=[
            pltpu.VMEM((maxblk, 1, _EBLK), jnp.int32),
            pltpu.VMEM((2, _EBLK, D), jnp.float32),
            pltpu.VMEM_SHARED((N, D), jnp.float32),
            pltpu.SemaphoreType.DMA,
            pltpu.SemaphoreType.DMA,
        ],
    )
    def k(ef_hbm, row_hbm, p0_hbm, p1_hbm, ridx, buf, agg, l0, l1):
        cid = lax.axis_index("c")
        sid = lax.axis_index("s")
        wid = sid * _NC + cid
        has_extra = wid < extra
        blk0 = wid * base + jnp.minimum(wid, extra)
        lsem = (l0, l1)

        def zrow(r, _):
            for c in range(D // _L):
                buf[0, r, pl.ds(c * _L, _L)] = jnp.zeros((_L,), jnp.float32)
            return 0

        lax.fori_loop(0, 128, zrow, 0)

        nz = zb_base + jnp.where(sid < zb_extra, 1, 0)

        def zcopy(j, _):
            blk = j * _NS + sid
            pltpu.sync_copy(buf.at[0], agg.at[pl.ds(blk * 128, 128)])
            return 0

        lax.fori_loop(0, nz, zcopy, 0)
        if nrow_tail:
            @pl.when(sid == _NS - 1)
            def _():
                pltpu.sync_copy(
                    buf.at[0].at[pl.ds(0, nrow_tail)],
                    agg.at[pl.ds(nrow_blk * 128, nrow_tail)],
                )
        plsc.subcore_barrier()

        pltpu.sync_copy(row_hbm.at[pl.ds(blk0, base)], ridx.at[pl.ds(0, base)])
        if extra:
            @pl.when(has_extra)
            def _():
                pltpu.sync_copy(
                    row_hbm.at[pl.ds(blk0 + base, 1)], ridx.at[pl.ds(base, 1)]
                )

        def start(j, p):
            pltpu.async_copy(
                ef_hbm.at[pl.ds((blk0 + j) * _EBLK, _EBLK)], buf.at[p], lsem[p]
            )

        def process(j, p):
            pltpu.make_async_copy(
                ef_hbm.at[pl.ds((blk0 + j) * _EBLK, _EBLK)], buf.at[p], lsem[p]
            ).wait()
            pltpu.sync_copy(buf.at[p], agg.at[ridx.at[j, 0]], add=True)

        start(0, 0)
        for j in range(1, base):
            start(j, j & 1)
            process(j - 1, (j - 1) & 1)
        if extra:
            @pl.when(has_extra)
            def _():
                start(base, base & 1)
        process(base - 1, (base - 1) & 1)
        if extra:
            @pl.when(has_extra)
            def _():
                process(base, base & 1)
        plsc.subcore_barrier()

        def wcopy(j, _):
            blk = j * _NS + sid
            sl = pl.ds(blk * 128, 128)

            @pl.when(cid == 0)
            def _():
                pltpu.sync_copy(agg.at[sl], p0_hbm.at[sl])

            @pl.when(cid == 1)
            def _():
                pltpu.sync_copy(agg.at[sl], p1_hbm.at[sl])

            return 0

        lax.fori_loop(0, nz, wcopy, 0)
        if nrow_tail:
            @pl.when(sid == _NS - 1)
            def _():
                tsl = pl.ds(nrow_blk * 128, nrow_tail)

                @pl.when(cid == 0)
                def _():
                    pltpu.sync_copy(agg.at[tsl], p0_hbm.at[tsl])

                @pl.when(cid == 1)
                def _():
                    pltpu.sync_copy(agg.at[tsl], p1_hbm.at[tsl])

    return k(ef, row3d)


def _node_body(h_ref, p0_ref, p1_ref, wna_ref, wnb_ref, bn1_ref, Wn2_ref,
               bn2_ref, Wl_ref, bl_ref, out_ref):
    hb = h_ref[...]
    agg = (p0_ref[...] + p1_ref[...]) * 0.01
    t = (
        jnp.dot(hb, wna_ref[...], preferred_element_type=jnp.float32)
        + jnp.dot(agg, wnb_ref[...], preferred_element_type=jnp.float32)
        + bn1_ref[...]
    )
    t = t * jax.nn.sigmoid(t)
    out = hb + jnp.dot(t, Wn2_ref[...], preferred_element_type=jnp.float32) + bn2_ref[...]
    out_ref[...] = (
        jnp.dot(out, Wl_ref[...], preferred_element_type=jnp.float32) + bl_ref[...]
    )


def _node_mlp(h, p0, p1, Wn1a, Wn1b, bn1, Wn2, bn2, Wl, bl):
    N, D = h.shape
    BN = 2000
    grid = (N // BN,)
    bspec_nd = pl.BlockSpec((BN, D), lambda i: (i, 0))
    bspec_w = pl.BlockSpec((D, D), lambda i: (0, 0))
    bspec_b = pl.BlockSpec((1, D), lambda i: (0, 0))
    return pl.pallas_call(
        _node_body,
        grid=grid,
        in_specs=[
            bspec_nd, bspec_nd, bspec_nd,
            bspec_w, bspec_w, bspec_b,
            bspec_w, bspec_b, bspec_w, bspec_b,
        ],
        out_specs=bspec_nd,
        out_shape=jax.ShapeDtypeStruct((N, D), jnp.float32),
    )(h, p0, p1, Wn1a, Wn1b, bn1, Wn2, bn2, Wl, bl)


def kernel(h, distances, edges, node_mask, edge_mask, h_gauss, W_lin, b_lin,
           We1, be1, We2, be2, Wn1, bn1, Wn2, bn2, Wa, ba):
    N, D = h.shape
    E = distances.shape[0]

    row = edges[0].astype(jnp.int32)
    col = edges[1].astype(jnp.int32)
    row3d = row.reshape(E // _EBLK, 1, _EBLK)
    col3d = col.reshape(E // _EBLK, 1, _EBLK)

    hh = jax.nn.softplus(h_gauss)[0]
    left = 1.0 / (math.sqrt(2.0 * math.pi) * hh)
    inv2 = 1.0 / (2.0 * hh * hh)
    scalars = jnp.stack([left, inv2]).reshape(1, 2)

    We1a = We1[:D]
    We1b = We1[D:2 * D]
    w256 = We1[2 * D:]
    be1_r = be1.reshape(1, D)
    be2_r = be2.reshape(1, D)
    bn1_r = bn1.reshape(1, D)
    bn2_r = bn2.reshape(1, D)
    bl_r = b_lin.reshape(1, D)
    ba_r = ba.reshape(1, 1)
    Wn1a = Wn1[:D]
    Wn1b = Wn1[D:]

    A, B = _prep(h, We1a, We1b, be1_r)
    pre = _gather_add(row3d, col3d, A, B, E, D)
    ef = _edge_mlp(scalars, pre, distances, edge_mask, w256,
                   We2.astype(jnp.bfloat16), be2_r, Wa, ba_r)
    p0, p1 = _scatter_add(ef, row3d, N, D)
    hidden = _node_mlp(h, p0, p1, Wn1a, Wn1b, bn1_r, Wn2, bn2_r, W_lin, bl_r)

    return (hidden, distances, edges, node_mask, edge_mask)


# speedup vs baseline: 1.1088x; 1.0288x over previous
import math
import functools

import jax
import jax.numpy as jnp
from jax import lax
from jax.experimental import pallas as pl
from jax.experimental.pallas import tpu as pltpu
from jax.experimental.pallas import tpu_sc as plsc

_NC = 2
_NS = 16
_NW = _NC * _NS
_L = 16

_EBLK = 128


def _prep_body(h_ref, wa_ref, wb_ref, be1_ref, A_ref, B_ref):
    hb = h_ref[...]
    A_ref[...] = (
        jnp.dot(hb, wa_ref[...], preferred_element_type=jnp.float32) + be1_ref[...]
    )
    B_ref[...] = jnp.dot(hb, wb_ref[...], preferred_element_type=jnp.float32)


def _prep(h, We1a, We1b, be1):
    N, D = h.shape
    BN = 2000
    grid = (N // BN,)
    return pl.pallas_call(
        _prep_body,
        grid=grid,
        in_specs=[
            pl.BlockSpec((BN, D), lambda i: (i, 0)),
            pl.BlockSpec((D, D), lambda i: (0, 0)),
            pl.BlockSpec((D, D), lambda i: (0, 0)),
            pl.BlockSpec((1, D), lambda i: (0, 0)),
        ],
        out_specs=[
            pl.BlockSpec((BN, D), lambda i: (i, 0)),
            pl.BlockSpec((BN, D), lambda i: (i, 0)),
        ],
        out_shape=[
            jax.ShapeDtypeStruct((N, D), jnp.float32),
            jax.ShapeDtypeStruct((N, D), jnp.float32),
        ],
    )(h, We1a, We1b, be1)


def _gather_add(row3d, col3d, A, B, E, D):
    nblk = E // _EBLK
    base = nblk // _NW
    extra = nblk % _NW
    maxblk = base + (1 if extra else 0)

    mesh = plsc.VectorSubcoreMesh(core_axis_name="c", subcore_axis_name="s")

    @functools.partial(
        pl.kernel,
        out_type=jax.ShapeDtypeStruct((E, D), jnp.float32),
        mesh=mesh,
        scratch_types=[
            pltpu.VMEM((maxblk, 1, _EBLK), jnp.int32),
            pltpu.VMEM((maxblk, 1, _EBLK), jnp.int32),
            pltpu.VMEM((2, _EBLK, D), jnp.float32),
            pltpu.VMEM((2, _EBLK, D), jnp.float32),
            pltpu.SemaphoreType.DMA,
            pltpu.SemaphoreType.DMA,
            pltpu.SemaphoreType.DMA,
            pltpu.SemaphoreType.DMA,
        ],
    )
    def k(row_hbm, col_hbm, A_hbm, B_hbm, pre_hbm, ridx, cidx, bufA, bufB,
          g0, g1, w0, w1):
        wid = lax.axis_index("s") * _NC + lax.axis_index("c")
        has_extra = wid < extra
        blk0 = wid * base + jnp.minimum(wid, extra)

        gsem = (g0, g1)
        wsem = (w0, w1)

        pltpu.sync_copy(row_hbm.at[pl.ds(blk0, base)], ridx.at[pl.ds(0, base)])
        pltpu.sync_copy(col_hbm.at[pl.ds(blk0, base)], cidx.at[pl.ds(0, base)])
        if extra:
            @pl.when(has_extra)
            def _():
                pltpu.sync_copy(
                    row_hbm.at[pl.ds(blk0 + base, 1)], ridx.at[pl.ds(base, 1)]
                )
                pltpu.sync_copy(
                    col_hbm.at[pl.ds(blk0 + base, 1)], cidx.at[pl.ds(base, 1)]
                )

        def start(j, p):
            pltpu.async_copy(A_hbm.at[ridx.at[j, 0]], bufA.at[p], gsem[p])
            pltpu.async_copy(B_hbm.at[cidx.at[j, 0]], bufB.at[p], gsem[p])

        def process(j, p):
            pltpu.make_async_copy(A_hbm.at[ridx.at[j, 0]], bufA.at[p], gsem[p]).wait()
            pltpu.make_async_copy(B_hbm.at[cidx.at[j, 0]], bufB.at[p], gsem[p]).wait()

            def row_body(r, _):
                for c in range(D // _L):
                    sl = pl.ds(c * _L, _L)
                    bufA[p, r, sl] = bufA[p, r, sl] + bufB[p, r, sl]
                return 0

            lax.fori_loop(0, _EBLK, row_body, 0)
            pltpu.async_copy(
                bufA.at[p], pre_hbm.at[pl.ds((blk0 + j) * _EBLK, _EBLK)], wsem[p]
            )

        def wait_write(j, p):
            pltpu.make_async_copy(
                bufA.at[p], pre_hbm.at[pl.ds((blk0 + j) * _EBLK, _EBLK)], wsem[p]
            ).wait()

        start(0, 0)
        start(1, 1)
        process(0, 0)
        for j in range(2, base):
            p = j & 1
            wait_write(j - 2, p)
            start(j, p)
            process(j - 1, 1 - p)
        if extra:
            p = base & 1
            @pl.when(has_extra)
            def _():
                wait_write(base - 2, p)
                start(base, p)
        process(base - 1, (base - 1) & 1)
        if extra:
            @pl.when(has_extra)
            def _():
                process(base, base & 1)
        wait_write(base - 1, (base - 1) & 1)
        if extra:
            @pl.when(has_extra)
            def _():
                wait_write(base, base & 1)

            @pl.when(jnp.logical_not(has_extra))
            def _():
                wait_write(base - 2, base & 1)
        else:
            wait_write(base - 2, base & 1)

    return k(row3d, col3d, A, B)


def _edge_body(sc_ref, pre_ref, d_ref, em_ref, w256_ref, We2_ref,
               be2_ref, Wa_ref, ba_ref, ef_ref):
    left = sc_ref[0, 0]
    inv2 = sc_ref[0, 1]
    d = d_ref[...]
    em = em_ref[...]
    g = left * jnp.exp(-(d * d) * inv2) * em
    x = pre_ref[...] + g * w256_ref[...]
    h1 = x * jax.nn.sigmoid(x)
    mij = jnp.dot(
        h1.astype(jnp.bfloat16), We2_ref[...],
        preferred_element_type=jnp.float32,
    ) + be2_ref[...]
    att = jax.nn.sigmoid(
        jnp.dot(mij, Wa_ref[...], preferred_element_type=jnp.float32) + ba_ref[...]
    )
    ef_ref[...] = mij * att * em


def _edge_mlp(scalars, pre, distances, edge_mask, w256, We2, be2, Wa, ba):
    E, D = pre.shape
    BE = 3200
    grid = (E // BE,)
    return pl.pallas_call(
        _edge_body,
        grid=grid,
        in_specs=[
            pl.BlockSpec(memory_space=pltpu.SMEM),
            pl.BlockSpec((BE, D), lambda i: (i, 0)),
            pl.BlockSpec((BE, 1), lambda i: (i, 0)),
            pl.BlockSpec((BE, 1), lambda i: (i, 0)),
            pl.BlockSpec((1, D), lambda i: (0, 0)),
            pl.BlockSpec((D, D), lambda i: (0, 0)),
            pl.BlockSpec((1, D), lambda i: (0, 0)),
            pl.BlockSpec((D, 1), lambda i: (0, 0)),
            pl.BlockSpec((1, 1), lambda i: (0, 0)),
        ],
        out_specs=pl.BlockSpec((BE, D), lambda i: (i, 0)),
        out_shape=jax.ShapeDtypeStruct((E, D), jnp.float32),
    )(scalars, pre, distances, edge_mask, w256, We2, be2, Wa, ba)


def _scatter_add(ef, row3d, N, D):
    E = ef.shape[0]
    nblk = E // _EBLK
    base = nblk // _NW
    extra = nblk % _NW
    maxblk = base + (1 if extra else 0)
    nrow_blk = N // 128
    nrow_tail = N - nrow_blk * 128
    zb_base = nrow_blk // _NS
    zb_extra = nrow_blk % _NS

    mesh = plsc.VectorSubcoreMesh(core_axis_name="c", subcore_axis_name="s")

    @functools.partial(
        pl.kernel,
        out_type=[
            jax.ShapeDtypeStruct((N, D), jnp.float32),
            jax.ShapeDtypeStruct((N, D), jnp.float32),
        ],
        mesh=mesh,
        scratch_types=[
            pltpu.VMEM((maxblk, 1, _EBLK), jnp.int32),
            pltpu.VMEM((2, _EBLK, D), jnp.float32),
            pltpu.VMEM_SHARED((N, D), jnp.float32),
            pltpu.SemaphoreType.DMA,
            pltpu.SemaphoreType.DMA,
        ],
    )
    def k(ef_hbm, row_hbm, p0_hbm, p1_hbm, ridx, buf, agg, l0, l1):
        cid = lax.axis_index("c")
        sid = lax.axis_index("s")
        wid = sid * _NC + cid
        has_extra = wid < extra
        blk0 = wid * base + jnp.minimum(wid, extra)
        lsem = (l0, l1)

        def zrow(r, _):
            for c in range(D // _L):
                buf[0, r, pl.ds(c * _L, _L)] = jnp.zeros((_L,), jnp.float32)
            return 0

        lax.fori_loop(0, 128, zrow, 0)

        nz = zb_base + jnp.where(sid < zb_extra, 1, 0)

        def zcopy(j, _):
            blk = j * _NS + sid
            pltpu.sync_copy(buf.at[0], agg.at[pl.ds(blk * 128, 128)])
            return 0

        lax.fori_loop(0, nz, zcopy, 0)
        if nrow_tail:
            @pl.when(sid == _NS - 1)
            def _():
                pltpu.sync_copy(
                    buf.at[0].at[pl.ds(0, nrow_tail)],
                    agg.at[pl.ds(nrow_blk * 128, nrow_tail)],
                )
        plsc.subcore_barrier()

        pltpu.sync_copy(row_hbm.at[pl.ds(blk0, base)], ridx.at[pl.ds(0, base)])
        if extra:
            @pl.when(has_extra)
            def _():
                pltpu.sync_copy(
                    row_hbm.at[pl.ds(blk0 + base, 1)], ridx.at[pl.ds(base, 1)]
                )

        def start(j, p):
            pltpu.async_copy(
                ef_hbm.at[pl.ds((blk0 + j) * _EBLK, _EBLK)], buf.at[p], lsem[p]
            )

        def process(j, p):
            pltpu.make_async_copy(
                ef_hbm.at[pl.ds((blk0 + j) * _EBLK, _EBLK)], buf.at[p], lsem[p]
            ).wait()
            pltpu.sync_copy(buf.at[p], agg.at[ridx.at[j, 0]], add=True)

        start(0, 0)
        for j in range(1, base):
            start(j, j & 1)
            process(j - 1, (j - 1) & 1)
        if extra:
            @pl.when(has_extra)
            def _():
                start(base, base & 1)
        process(base - 1, (base - 1) & 1)
        if extra:
            @pl.when(has_extra)
            def _():
                process(base, base & 1)
        plsc.subcore_barrier()

        def wcopy(j, _):
            blk = j * _NS + sid
            sl = pl.ds(blk * 128, 128)

            @pl.when(cid == 0)
            def _():
                pltpu.sync_copy(agg.at[sl], p0_hbm.at[sl])

            @pl.when(cid == 1)
            def _():
                pltpu.sync_copy(agg.at[sl], p1_hbm.at[sl])

            return 0

        lax.fori_loop(0, nz, wcopy, 0)
        if nrow_tail:
            @pl.when(sid == _NS - 1)
            def _():
                tsl = pl.ds(nrow_blk * 128, nrow_tail)

                @pl.when(cid == 0)
                def _():
                    pltpu.sync_copy(agg.at[tsl], p0_hbm.at[tsl])

                @pl.when(cid == 1)
                def _():
                    pltpu.sync_copy(agg.at[tsl], p1_hbm.at[tsl])

    return k(ef, row3d)


def _node_body(h_ref, p0_ref, p1_ref, p2_ref, p3_ref, wna_ref, wnb_ref,
               bn1_ref, Wn2_ref, bn2_ref, Wl_ref, bl_ref, out_ref):
    hb = h_ref[...]
    agg = (p0_ref[...] + p1_ref[...] + p2_ref[...] + p3_ref[...]) * 0.01
    t = (
        jnp.dot(hb, wna_ref[...], preferred_element_type=jnp.float32)
        + jnp.dot(agg, wnb_ref[...], preferred_element_type=jnp.float32)
        + bn1_ref[...]
    )
    t = t * jax.nn.sigmoid(t)
    out = hb + jnp.dot(t, Wn2_ref[...], preferred_element_type=jnp.float32) + bn2_ref[...]
    out_ref[...] = (
        jnp.dot(out, Wl_ref[...], preferred_element_type=jnp.float32) + bl_ref[...]
    )


def _node_mlp(h, parts, Wn1a, Wn1b, bn1, Wn2, bn2, Wl, bl):
    N, D = h.shape
    BN = 2000
    grid = (N // BN,)
    bspec_nd = pl.BlockSpec((BN, D), lambda i: (i, 0))
    bspec_w = pl.BlockSpec((D, D), lambda i: (0, 0))
    bspec_b = pl.BlockSpec((1, D), lambda i: (0, 0))
    return pl.pallas_call(
        _node_body,
        grid=grid,
        in_specs=[
            bspec_nd, bspec_nd, bspec_nd, bspec_nd, bspec_nd,
            bspec_w, bspec_w, bspec_b,
            bspec_w, bspec_b, bspec_w, bspec_b,
        ],
        out_specs=bspec_nd,
        out_shape=jax.ShapeDtypeStruct((N, D), jnp.float32),
    )(h, *parts, Wn1a, Wn1b, bn1, Wn2, bn2, Wl, bl)


def kernel(h, distances, edges, node_mask, edge_mask, h_gauss, W_lin, b_lin,
           We1, be1, We2, be2, Wn1, bn1, Wn2, bn2, Wa, ba):
    N, D = h.shape
    E = distances.shape[0]

    row = edges[0].astype(jnp.int32)
    col = edges[1].astype(jnp.int32)
    row3d = row.reshape(E // _EBLK, 1, _EBLK)
    col3d = col.reshape(E // _EBLK, 1, _EBLK)
    EH = E // 2
    HBLK = EH // _EBLK

    hh = jax.nn.softplus(h_gauss)[0]
    left = 1.0 / (math.sqrt(2.0 * math.pi) * hh)
    inv2 = 1.0 / (2.0 * hh * hh)
    scalars = jnp.stack([left, inv2]).reshape(1, 2)

    We1a = We1[:D]
    We1b = We1[D:2 * D]
    w256 = We1[2 * D:]
    be1_r = be1.reshape(1, D)
    be2_r = be2.reshape(1, D)
    bn1_r = bn1.reshape(1, D)
    bn2_r = bn2.reshape(1, D)
    bl_r = b_lin.reshape(1, D)
    ba_r = ba.reshape(1, 1)
    Wn1a = Wn1[:D]
    Wn1b = Wn1[D:]

    A, B = _prep(h, We1a, We1b, be1_r)

    We2b = We2.astype(jnp.bfloat16)
    parts = []
    efs = []
    for half in range(2):
        r3 = row3d[half * HBLK:(half + 1) * HBLK]
        c3 = col3d[half * HBLK:(half + 1) * HBLK]
        pre = _gather_add(r3, c3, A, B, EH, D)
        ef = _edge_mlp(scalars, pre,
                       distances[half * EH:(half + 1) * EH],
                       edge_mask[half * EH:(half + 1) * EH],
                       w256, We2b, be2_r, Wa, ba_r)
        efs.append((ef, r3))
    for ef, r3 in efs:
        p0, p1 = _scatter_add(ef, r3, N, D)
        parts.extend([p0, p1])
    hidden = _node_mlp(h, parts, Wn1a, Wn1b, bn1_r, Wn2, bn2_r, W_lin, bl_r)

    return (hidden, distances, edges, node_mask, edge_mask)


# speedup vs baseline: 1.1331x; 1.0219x over previous
import math
import functools

import jax
import jax.numpy as jnp
from jax import lax
from jax.experimental import pallas as pl
from jax.experimental.pallas import tpu as pltpu
from jax.experimental.pallas import tpu_sc as plsc

_NC = 2
_NS = 16
_NW = _NC * _NS
_L = 16

_EBLK = 128


def _prep_body(h_ref, wa_ref, wb_ref, be1_ref, A_ref, B_ref):
    hb = h_ref[...]
    A_ref[...] = (
        jnp.dot(hb, wa_ref[...], preferred_element_type=jnp.float32) + be1_ref[...]
    )
    B_ref[...] = jnp.dot(hb, wb_ref[...], preferred_element_type=jnp.float32)


def _prep(h, We1a, We1b, be1):
    N, D = h.shape
    BN = 2000
    grid = (N // BN,)
    return pl.pallas_call(
        _prep_body,
        grid=grid,
        in_specs=[
            pl.BlockSpec((BN, D), lambda i: (i, 0)),
            pl.BlockSpec((D, D), lambda i: (0, 0)),
            pl.BlockSpec((D, D), lambda i: (0, 0)),
            pl.BlockSpec((1, D), lambda i: (0, 0)),
        ],
        out_specs=[
            pl.BlockSpec((BN, D), lambda i: (i, 0)),
            pl.BlockSpec((BN, D), lambda i: (i, 0)),
        ],
        out_shape=[
            jax.ShapeDtypeStruct((N, D), jnp.float32),
            jax.ShapeDtypeStruct((N, D), jnp.float32),
        ],
    )(h, We1a, We1b, be1)


def _gather_add(row3d, col3d, A, B, E, D):
    nblk = E // _EBLK
    base = nblk // _NW
    extra = nblk % _NW
    maxblk = base + (1 if extra else 0)

    mesh = plsc.VectorSubcoreMesh(core_axis_name="c", subcore_axis_name="s")

    @functools.partial(
        pl.kernel,
        out_type=jax.ShapeDtypeStruct((E, D), jnp.float32),
        mesh=mesh,
        scratch_types=[
            pltpu.VMEM((maxblk, 1, _EBLK), jnp.int32),
            pltpu.VMEM((maxblk, 1, _EBLK), jnp.int32),
            pltpu.VMEM((2, _EBLK, D), jnp.float32),
            pltpu.VMEM((2, _EBLK, D), jnp.float32),
            pltpu.SemaphoreType.DMA,
            pltpu.SemaphoreType.DMA,
            pltpu.SemaphoreType.DMA,
            pltpu.SemaphoreType.DMA,
        ],
    )
    def k(row_hbm, col_hbm, A_hbm, B_hbm, pre_hbm, ridx, cidx, bufA, bufB,
          g0, g1, w0, w1):
        wid = lax.axis_index("s") * _NC + lax.axis_index("c")
        has_extra = wid < extra
        blk0 = wid * base + jnp.minimum(wid, extra)

        gsem = (g0, g1)
        wsem = (w0, w1)

        pltpu.sync_copy(row_hbm.at[pl.ds(blk0, base)], ridx.at[pl.ds(0, base)])
        pltpu.sync_copy(col_hbm.at[pl.ds(blk0, base)], cidx.at[pl.ds(0, base)])
        if extra:
            @pl.when(has_extra)
            def _():
                pltpu.sync_copy(
                    row_hbm.at[pl.ds(blk0 + base, 1)], ridx.at[pl.ds(base, 1)]
                )
                pltpu.sync_copy(
                    col_hbm.at[pl.ds(blk0 + base, 1)], cidx.at[pl.ds(base, 1)]
                )

        def start(j, p):
            pltpu.async_copy(A_hbm.at[ridx.at[j, 0]], bufA.at[p], gsem[p])
            pltpu.async_copy(B_hbm.at[cidx.at[j, 0]], bufB.at[p], gsem[p])

        def process(j, p):
            pltpu.make_async_copy(A_hbm.at[ridx.at[j, 0]], bufA.at[p], gsem[p]).wait()
            pltpu.make_async_copy(B_hbm.at[cidx.at[j, 0]], bufB.at[p], gsem[p]).wait()

            def row_body(r, _):
                for c in range(D // _L):
                    sl = pl.ds(c * _L, _L)
                    bufA[p, r, sl] = bufA[p, r, sl] + bufB[p, r, sl]
                return 0

            lax.fori_loop(0, _EBLK, row_body, 0)
            pltpu.async_copy(
                bufA.at[p], pre_hbm.at[pl.ds((blk0 + j) * _EBLK, _EBLK)], wsem[p]
            )

        def wait_write(j, p):
            pltpu.make_async_copy(
                bufA.at[p], pre_hbm.at[pl.ds((blk0 + j) * _EBLK, _EBLK)], wsem[p]
            ).wait()

        start(0, 0)
        start(1, 1)
        process(0, 0)
        for j in range(2, base):
            p = j & 1
            wait_write(j - 2, p)
            start(j, p)
            process(j - 1, 1 - p)
        if extra:
            p = base & 1
            @pl.when(has_extra)
            def _():
                wait_write(base - 2, p)
                start(base, p)
        process(base - 1, (base - 1) & 1)
        if extra:
            @pl.when(has_extra)
            def _():
                process(base, base & 1)
        wait_write(base - 1, (base - 1) & 1)
        if extra:
            @pl.when(has_extra)
            def _():
                wait_write(base, base & 1)

            @pl.when(jnp.logical_not(has_extra))
            def _():
                wait_write(base - 2, base & 1)
        else:
            wait_write(base - 2, base & 1)

    return k(row3d, col3d, A, B)


def _edge_body(sc_ref, pre_ref, d_ref, em_ref, w256_ref, We2_ref,
               be2_ref, Wa_ref, ba_ref, ef_ref):
    left = sc_ref[0, 0]
    inv2 = sc_ref[0, 1]
    d = d_ref[...]
    em = em_ref[...]
    g = left * jnp.exp(-(d * d) * inv2) * em
    x = pre_ref[...] + g * w256_ref[...]
    h1 = x * jax.nn.sigmoid(x)
    mij = jnp.dot(
        h1.astype(jnp.bfloat16), We2_ref[...],
        preferred_element_type=jnp.float32,
    ) + be2_ref[...]
    att = jax.nn.sigmoid(
        jnp.dot(mij, Wa_ref[...], preferred_element_type=jnp.float32) + ba_ref[...]
    )
    ef_ref[...] = mij * att * em


def _edge_mlp(scalars, pre, distances, edge_mask, w256, We2, be2, Wa, ba):
    E, D = pre.shape
    BE = 6400
    grid = (E // BE,)
    return pl.pallas_call(
        _edge_body,
        grid=grid,
        in_specs=[
            pl.BlockSpec(memory_space=pltpu.SMEM),
            pl.BlockSpec((BE, D), lambda i: (i, 0)),
            pl.BlockSpec((BE, 1), lambda i: (i, 0)),
            pl.BlockSpec((BE, 1), lambda i: (i, 0)),
            pl.BlockSpec((1, D), lambda i: (0, 0)),
            pl.BlockSpec((D, D), lambda i: (0, 0)),
            pl.BlockSpec((1, D), lambda i: (0, 0)),
            pl.BlockSpec((D, 1), lambda i: (0, 0)),
            pl.BlockSpec((1, 1), lambda i: (0, 0)),
        ],
        out_specs=pl.BlockSpec((BE, D), lambda i: (i, 0)),
        out_shape=jax.ShapeDtypeStruct((E, D), jnp.float32),
    )(scalars, pre, distances, edge_mask, w256, We2, be2, Wa, ba)


def _scatter_add(ef, row3d, N, D):
    E = ef.shape[0]
    nblk = E // _EBLK
    base = nblk // _NW
    extra = nblk % _NW
    maxblk = base + (1 if extra else 0)
    nrow_blk = N // 128
    nrow_tail = N - nrow_blk * 128
    zb_base = nrow_blk // _NS
    zb_extra = nrow_blk % _NS

    mesh = plsc.VectorSubcoreMesh(core_axis_name="c", subcore_axis_name="s")

    @functools.partial(
        pl.kernel,
        out_type=[
            jax.ShapeDtypeStruct((N, D), jnp.float32),
            jax.ShapeDtypeStruct((N, D), jnp.float32),
        ],
        mesh=mesh,
        scratch_types=[
            pltpu.VMEM((maxblk, 1, _EBLK), jnp.int32),
            pltpu.VMEM((2, _EBLK, D), jnp.float32),
            pltpu.VMEM_SHARED((N, D), jnp.float32),
            pltpu.SemaphoreType.DMA,
            pltpu.SemaphoreType.DMA,
        ],
    )
    def k(ef_hbm, row_hbm, p0_hbm, p1_hbm, ridx, buf, agg, l0, l1):
        cid = lax.axis_index("c")
        sid = lax.axis_index("s")
        wid = sid * _NC + cid
        has_extra = wid < extra
        blk0 = wid * base + jnp.minimum(wid, extra)
        lsem = (l0, l1)

        def zrow(r, _):
            for c in range(D // _L):
                buf[0, r, pl.ds(c * _L, _L)] = jnp.zeros((_L,), jnp.float32)
            return 0

        lax.fori_loop(0, 128, zrow, 0)

        nz = zb_base + jnp.where(sid < zb_extra, 1, 0)

        def zcopy(j, _):
            blk = j * _NS + sid
            pltpu.sync_copy(buf.at[0], agg.at[pl.ds(blk * 128, 128)])
            return 0

        lax.fori_loop(0, nz, zcopy, 0)
        if nrow_tail:
            @pl.when(sid == _NS - 1)
            def _():
                pltpu.sync_copy(
                    buf.at[0].at[pl.ds(0, nrow_tail)],
                    agg.at[pl.ds(nrow_blk * 128, nrow_tail)],
                )
        plsc.subcore_barrier()

        pltpu.sync_copy(row_hbm.at[pl.ds(blk0, base)], ridx.at[pl.ds(0, base)])
        if extra:
            @pl.when(has_extra)
            def _():
                pltpu.sync_copy(
                    row_hbm.at[pl.ds(blk0 + base, 1)], ridx.at[pl.ds(base, 1)]
                )

        def start(j, p):
            pltpu.async_copy(
                ef_hbm.at[pl.ds((blk0 + j) * _EBLK, _EBLK)], buf.at[p], lsem[p]
            )

        def process(j, p):
            pltpu.make_async_copy(
                ef_hbm.at[pl.ds((blk0 + j) * _EBLK, _EBLK)], buf.at[p], lsem[p]
            ).wait()
            pltpu.sync_copy(buf.at[p], agg.at[ridx.at[j, 0]], add=True)

        start(0, 0)
        for j in range(1, base):
            start(j, j & 1)
            process(j - 1, (j - 1) & 1)
        if extra:
            @pl.when(has_extra)
            def _():
                start(base, base & 1)
        process(base - 1, (base - 1) & 1)
        if extra:
            @pl.when(has_extra)
            def _():
                process(base, base & 1)
        plsc.subcore_barrier()

        def wcopy(j, _):
            blk = j * _NS + sid
            sl = pl.ds(blk * 128, 128)

            @pl.when(cid == 0)
            def _():
                pltpu.sync_copy(agg.at[sl], p0_hbm.at[sl])

            @pl.when(cid == 1)
            def _():
                pltpu.sync_copy(agg.at[sl], p1_hbm.at[sl])

            return 0

        lax.fori_loop(0, nz, wcopy, 0)
        if nrow_tail:
            @pl.when(sid == _NS - 1)
            def _():
                tsl = pl.ds(nrow_blk * 128, nrow_tail)

                @pl.when(cid == 0)
                def _():
                    pltpu.sync_copy(agg.at[tsl], p0_hbm.at[tsl])

                @pl.when(cid == 1)
                def _():
                    pltpu.sync_copy(agg.at[tsl], p1_hbm.at[tsl])

    return k(ef, row3d)


def _node_body(h_ref, p0_ref, p1_ref, p2_ref, p3_ref, wna_ref, wnb_ref,
               bn1_ref, Wn2_ref, bn2_ref, Wl_ref, bl_ref, out_ref):
    hb = h_ref[...]
    agg = (p0_ref[...] + p1_ref[...] + p2_ref[...] + p3_ref[...]) * 0.01
    t = (
        jnp.dot(hb, wna_ref[...], preferred_element_type=jnp.float32)
        + jnp.dot(agg, wnb_ref[...], preferred_element_type=jnp.float32)
        + bn1_ref[...]
    )
    t = t * jax.nn.sigmoid(t)
    out = hb + jnp.dot(t, Wn2_ref[...], preferred_element_type=jnp.float32) + bn2_ref[...]
    out_ref[...] = (
        jnp.dot(out, Wl_ref[...], preferred_element_type=jnp.float32) + bl_ref[...]
    )


def _node_mlp(h, parts, Wn1a, Wn1b, bn1, Wn2, bn2, Wl, bl):
    N, D = h.shape
    BN = 2000
    grid = (N // BN,)
    bspec_nd = pl.BlockSpec((BN, D), lambda i: (i, 0))
    bspec_w = pl.BlockSpec((D, D), lambda i: (0, 0))
    bspec_b = pl.BlockSpec((1, D), lambda i: (0, 0))
    return pl.pallas_call(
        _node_body,
        grid=grid,
        in_specs=[
            bspec_nd, bspec_nd, bspec_nd, bspec_nd, bspec_nd,
            bspec_w, bspec_w, bspec_b,
            bspec_w, bspec_b, bspec_w, bspec_b,
        ],
        out_specs=bspec_nd,
        out_shape=jax.ShapeDtypeStruct((N, D), jnp.float32),
    )(h, *parts, Wn1a, Wn1b, bn1, Wn2, bn2, Wl, bl)


def kernel(h, distances, edges, node_mask, edge_mask, h_gauss, W_lin, b_lin,
           We1, be1, We2, be2, Wn1, bn1, Wn2, bn2, Wa, ba):
    N, D = h.shape
    E = distances.shape[0]

    row = edges[0].astype(jnp.int32)
    col = edges[1].astype(jnp.int32)
    row3d = row.reshape(E // _EBLK, 1, _EBLK)
    col3d = col.reshape(E // _EBLK, 1, _EBLK)
    EH = E // 2
    HBLK = EH // _EBLK

    hh = jax.nn.softplus(h_gauss)[0]
    left = 1.0 / (math.sqrt(2.0 * math.pi) * hh)
    inv2 = 1.0 / (2.0 * hh * hh)
    scalars = jnp.stack([left, inv2]).reshape(1, 2)

    We1a = We1[:D]
    We1b = We1[D:2 * D]
    w256 = We1[2 * D:]
    be1_r = be1.reshape(1, D)
    be2_r = be2.reshape(1, D)
    bn1_r = bn1.reshape(1, D)
    bn2_r = bn2.reshape(1, D)
    bl_r = b_lin.reshape(1, D)
    ba_r = ba.reshape(1, 1)
    Wn1a = Wn1[:D]
    Wn1b = Wn1[D:]

    A, B = _prep(h, We1a, We1b, be1_r)

    We2b = We2.astype(jnp.bfloat16)
    parts = []
    efs = []
    for half in range(2):
        r3 = row3d[half * HBLK:(half + 1) * HBLK]
        c3 = col3d[half * HBLK:(half + 1) * HBLK]
        pre = _gather_add(r3, c3, A, B, EH, D)
        ef = _edge_mlp(scalars, pre,
                       distances[half * EH:(half + 1) * EH],
                       edge_mask[half * EH:(half + 1) * EH],
                       w256, We2b, be2_r, Wa, ba_r)
        efs.append((ef, r3))
    for ef, r3 in efs:
        p0, p1 = _scatter_add(ef, r3, N, D)
        parts.extend([p0, p1])
    hidden = _node_mlp(h, parts, Wn1a, Wn1b, bn1_r, Wn2, bn2_r, W_lin, bl_r)

    return (hidden, distances, edges, node_mask, edge_mask)


# speedup vs baseline: 1.1451x; 1.0106x over previous
import math
import functools

import jax
import jax.numpy as jnp
from jax import lax
from jax.experimental import pallas as pl
from jax.experimental.pallas import tpu as pltpu
from jax.experimental.pallas import tpu_sc as plsc

_NC = 2
_NS = 16
_NW = _NC * _NS
_L = 16

_EBLK = 128


def _prep_body(h_ref, wa_ref, wb_ref, be1_ref, A_ref, B_ref):
    hb = h_ref[...]
    A_ref[...] = (
        jnp.dot(hb, wa_ref[...], preferred_element_type=jnp.float32) + be1_ref[...]
    )
    B_ref[...] = jnp.dot(hb, wb_ref[...], preferred_element_type=jnp.float32)


def _prep(h, We1a, We1b, be1):
    N, D = h.shape
    BN = 2000
    grid = (N // BN,)
    return pl.pallas_call(
        _prep_body,
        grid=grid,
        in_specs=[
            pl.BlockSpec((BN, D), lambda i: (i, 0)),
            pl.BlockSpec((D, D), lambda i: (0, 0)),
            pl.BlockSpec((D, D), lambda i: (0, 0)),
            pl.BlockSpec((1, D), lambda i: (0, 0)),
        ],
        out_specs=[
            pl.BlockSpec((BN, D), lambda i: (i, 0)),
            pl.BlockSpec((BN, D), lambda i: (i, 0)),
        ],
        out_shape=[
            jax.ShapeDtypeStruct((N, D), jnp.float32),
            jax.ShapeDtypeStruct((N, D), jnp.float32),
        ],
    )(h, We1a, We1b, be1)


def _gather_add(row3d, col3d, A, B, E, D):
    nblk = E // _EBLK
    base = nblk // _NW
    extra = nblk % _NW
    maxblk = base + (1 if extra else 0)

    mesh = plsc.VectorSubcoreMesh(core_axis_name="c", subcore_axis_name="s")

    @functools.partial(
        pl.kernel,
        out_type=jax.ShapeDtypeStruct((E, D), jnp.float32),
        mesh=mesh,
        scratch_types=[
            pltpu.VMEM((maxblk, 1, _EBLK), jnp.int32),
            pltpu.VMEM((maxblk, 1, _EBLK), jnp.int32),
            pltpu.VMEM((2, _EBLK, D), jnp.float32),
            pltpu.VMEM((2, _EBLK, D), jnp.float32),
            pltpu.SemaphoreType.DMA,
            pltpu.SemaphoreType.DMA,
            pltpu.SemaphoreType.DMA,
            pltpu.SemaphoreType.DMA,
        ],
    )
    def k(row_hbm, col_hbm, A_hbm, B_hbm, pre_hbm, ridx, cidx, bufA, bufB,
          g0, g1, w0, w1):
        wid = lax.axis_index("s") * _NC + lax.axis_index("c")
        has_extra = wid < extra
        blk0 = wid * base + jnp.minimum(wid, extra)

        gsem = (g0, g1)
        wsem = (w0, w1)

        pltpu.sync_copy(row_hbm.at[pl.ds(blk0, base)], ridx.at[pl.ds(0, base)])
        pltpu.sync_copy(col_hbm.at[pl.ds(blk0, base)], cidx.at[pl.ds(0, base)])
        if extra:
            @pl.when(has_extra)
            def _():
                pltpu.sync_copy(
                    row_hbm.at[pl.ds(blk0 + base, 1)], ridx.at[pl.ds(base, 1)]
                )
                pltpu.sync_copy(
                    col_hbm.at[pl.ds(blk0 + base, 1)], cidx.at[pl.ds(base, 1)]
                )

        def start(j, p):
            pltpu.async_copy(A_hbm.at[ridx.at[j, 0]], bufA.at[p], gsem[p])
            pltpu.async_copy(B_hbm.at[cidx.at[j, 0]], bufB.at[p], gsem[p])

        def process(j, p):
            pltpu.make_async_copy(A_hbm.at[ridx.at[j, 0]], bufA.at[p], gsem[p]).wait()
            pltpu.make_async_copy(B_hbm.at[cidx.at[j, 0]], bufB.at[p], gsem[p]).wait()

            def row_body(r, _):
                for c in range(D // _L):
                    sl = pl.ds(c * _L, _L)
                    bufA[p, r, sl] = bufA[p, r, sl] + bufB[p, r, sl]
                return 0

            lax.fori_loop(0, _EBLK, row_body, 0)
            pltpu.async_copy(
                bufA.at[p], pre_hbm.at[pl.ds((blk0 + j) * _EBLK, _EBLK)], wsem[p]
            )

        def wait_write(j, p):
            pltpu.make_async_copy(
                bufA.at[p], pre_hbm.at[pl.ds((blk0 + j) * _EBLK, _EBLK)], wsem[p]
            ).wait()

        start(0, 0)
        start(1, 1)
        process(0, 0)
        for j in range(2, base):
            p = j & 1
            wait_write(j - 2, p)
            start(j, p)
            process(j - 1, 1 - p)
        if extra:
            p = base & 1
            @pl.when(has_extra)
            def _():
                wait_write(base - 2, p)
                start(base, p)
        process(base - 1, (base - 1) & 1)
        if extra:
            @pl.when(has_extra)
            def _():
                process(base, base & 1)
        wait_write(base - 1, (base - 1) & 1)
        if extra:
            @pl.when(has_extra)
            def _():
                wait_write(base, base & 1)

            @pl.when(jnp.logical_not(has_extra))
            def _():
                wait_write(base - 2, base & 1)
        else:
            wait_write(base - 2, base & 1)

    return k(row3d, col3d, A, B)


def _edge_body(sc_ref, pre_ref, d_ref, em_ref, w256_ref, We2_ref,
               be2_ref, Wa_ref, ba_ref, ef_ref):
    left = sc_ref[0, 0]
    inv2 = sc_ref[0, 1]
    d = d_ref[...]
    em = em_ref[...]
    g = left * jnp.exp(-(d * d) * inv2) * em
    x = pre_ref[...] + g * w256_ref[...]
    h1 = x * jax.nn.sigmoid(x)
    mij = jnp.dot(
        h1.astype(jnp.bfloat16), We2_ref[...],
        preferred_element_type=jnp.float32,
    ) + be2_ref[...]
    att = jax.nn.sigmoid(
        jnp.dot(mij, Wa_ref[...], preferred_element_type=jnp.float32) + ba_ref[...]
    )
    ef_ref[...] = mij * att * em


def _edge_mlp(scalars, pre, distances, edge_mask, w256, We2, be2, Wa, ba):
    E, D = pre.shape
    BE = 6400
    grid = (E // BE,)
    return pl.pallas_call(
        _edge_body,
        grid=grid,
        in_specs=[
            pl.BlockSpec(memory_space=pltpu.SMEM),
            pl.BlockSpec((BE, D), lambda i: (i, 0)),
            pl.BlockSpec((BE, 1), lambda i: (i, 0)),
            pl.BlockSpec((BE, 1), lambda i: (i, 0)),
            pl.BlockSpec((1, D), lambda i: (0, 0)),
            pl.BlockSpec((D, D), lambda i: (0, 0)),
            pl.BlockSpec((1, D), lambda i: (0, 0)),
            pl.BlockSpec((D, 1), lambda i: (0, 0)),
            pl.BlockSpec((1, 1), lambda i: (0, 0)),
        ],
        out_specs=pl.BlockSpec((BE, D), lambda i: (i, 0)),
        out_shape=jax.ShapeDtypeStruct((E, D), jnp.float32),
    )(scalars, pre, distances, edge_mask, w256, We2, be2, Wa, ba)


def _scatter_add(ef, row3d, N, D):
    E = ef.shape[0]
    nblk = E // _EBLK
    base = nblk // _NW
    extra = nblk % _NW
    maxblk = base + (1 if extra else 0)
    nrow_blk = N // 128
    nrow_tail = N - nrow_blk * 128
    zb_base = nrow_blk // _NS
    zb_extra = nrow_blk % _NS

    mesh = plsc.VectorSubcoreMesh(core_axis_name="c", subcore_axis_name="s")

    @functools.partial(
        pl.kernel,
        out_type=[
            jax.ShapeDtypeStruct((N, D), jnp.float32),
            jax.ShapeDtypeStruct((N, D), jnp.float32),
        ],
        mesh=mesh,
        scratch_types=[
            pltpu.VMEM((maxblk, 1, _EBLK), jnp.int32),
            pltpu.VMEM((2, _EBLK, D), jnp.float32),
            pltpu.VMEM_SHARED((N, D), jnp.float32),
            pltpu.SemaphoreType.DMA,
            pltpu.SemaphoreType.DMA,
        ],
    )
    def k(ef_hbm, row_hbm, p0_hbm, p1_hbm, ridx, buf, agg, l0, l1):
        cid = lax.axis_index("c")
        sid = lax.axis_index("s")
        wid = sid * _NC + cid
        has_extra = wid < extra
        blk0 = wid * base + jnp.minimum(wid, extra)
        lsem = (l0, l1)

        def zrow(r, _):
            for c in range(D // _L):
                buf[0, r, pl.ds(c * _L, _L)] = jnp.zeros((_L,), jnp.float32)
            return 0

        lax.fori_loop(0, 128, zrow, 0)

        nz = zb_base + jnp.where(sid < zb_extra, 1, 0)

        def zcopy(j, _):
            blk = j * _NS + sid
            pltpu.sync_copy(buf.at[0], agg.at[pl.ds(blk * 128, 128)])
            return 0

        lax.fori_loop(0, nz, zcopy, 0)
        if nrow_tail:
            @pl.when(sid == _NS - 1)
            def _():
                pltpu.sync_copy(
                    buf.at[0].at[pl.ds(0, nrow_tail)],
                    agg.at[pl.ds(nrow_blk * 128, nrow_tail)],
                )
        plsc.subcore_barrier()

        pltpu.sync_copy(row_hbm.at[pl.ds(blk0, base)], ridx.at[pl.ds(0, base)])
        if extra:
            @pl.when(has_extra)
            def _():
                pltpu.sync_copy(
                    row_hbm.at[pl.ds(blk0 + base, 1)], ridx.at[pl.ds(base, 1)]
                )

        def start(j, p):
            pltpu.async_copy(
                ef_hbm.at[pl.ds((blk0 + j) * _EBLK, _EBLK)], buf.at[p], lsem[p]
            )

        def process(j, p):
            pltpu.make_async_copy(
                ef_hbm.at[pl.ds((blk0 + j) * _EBLK, _EBLK)], buf.at[p], lsem[p]
            ).wait()
            pltpu.sync_copy(buf.at[p], agg.at[ridx.at[j, 0]], add=True)

        start(0, 0)
        for j in range(1, base):
            start(j, j & 1)
            process(j - 1, (j - 1) & 1)
        if extra:
            @pl.when(has_extra)
            def _():
                start(base, base & 1)
        process(base - 1, (base - 1) & 1)
        if extra:
            @pl.when(has_extra)
            def _():
                process(base, base & 1)
        plsc.subcore_barrier()

        def wcopy(j, _):
            blk = j * _NS + sid
            sl = pl.ds(blk * 128, 128)

            @pl.when(cid == 0)
            def _():
                pltpu.sync_copy(agg.at[sl], p0_hbm.at[sl])

            @pl.when(cid == 1)
            def _():
                pltpu.sync_copy(agg.at[sl], p1_hbm.at[sl])

            return 0

        lax.fori_loop(0, nz, wcopy, 0)
        if nrow_tail:
            @pl.when(sid == _NS - 1)
            def _():
                tsl = pl.ds(nrow_blk * 128, nrow_tail)

                @pl.when(cid == 0)
                def _():
                    pltpu.sync_copy(agg.at[tsl], p0_hbm.at[tsl])

                @pl.when(cid == 1)
                def _():
                    pltpu.sync_copy(agg.at[tsl], p1_hbm.at[tsl])

    return k(ef, row3d)


def _node_body(h_ref, p0_ref, p1_ref, p2_ref, p3_ref, wna_ref, wnb_ref,
               bn1_ref, Wn2_ref, bn2_ref, Wl_ref, bl_ref, out_ref):
    hb = h_ref[...]
    agg = (p0_ref[...] + p1_ref[...] + p2_ref[...] + p3_ref[...]) * 0.01
    t = (
        jnp.dot(hb, wna_ref[...], preferred_element_type=jnp.float32)
        + jnp.dot(agg, wnb_ref[...], preferred_element_type=jnp.float32)
        + bn1_ref[...]
    )
    t = t * jax.nn.sigmoid(t)
    out = hb + jnp.dot(t, Wn2_ref[...], preferred_element_type=jnp.float32) + bn2_ref[...]
    out_ref[...] = (
        jnp.dot(out, Wl_ref[...], preferred_element_type=jnp.float32) + bl_ref[...]
    )


def _node_mlp(h, parts, Wn1a, Wn1b, bn1, Wn2, bn2, Wl, bl):
    N, D = h.shape
    BN = 2000
    grid = (N // BN,)
    bspec_nd = pl.BlockSpec((BN, D), lambda i: (i, 0))
    bspec_w = pl.BlockSpec((D, D), lambda i: (0, 0))
    bspec_b = pl.BlockSpec((1, D), lambda i: (0, 0))
    return pl.pallas_call(
        _node_body,
        grid=grid,
        in_specs=[
            bspec_nd, bspec_nd, bspec_nd, bspec_nd, bspec_nd,
            bspec_w, bspec_w, bspec_b,
            bspec_w, bspec_b, bspec_w, bspec_b,
        ],
        out_specs=bspec_nd,
        out_shape=jax.ShapeDtypeStruct((N, D), jnp.float32),
    )(h, *parts, Wn1a, Wn1b, bn1, Wn2, bn2, Wl, bl)


def kernel(h, distances, edges, node_mask, edge_mask, h_gauss, W_lin, b_lin,
           We1, be1, We2, be2, Wn1, bn1, Wn2, bn2, Wa, ba):
    N, D = h.shape
    E = distances.shape[0]

    row = edges[0].astype(jnp.int32)
    col = edges[1].astype(jnp.int32)
    row3d = row.reshape(E // _EBLK, 1, _EBLK)
    col3d = col.reshape(E // _EBLK, 1, _EBLK)
    E1 = (3 * E // 5 // 6400) * 6400
    bounds = [(0, E1), (E1, E)]

    hh = jax.nn.softplus(h_gauss)[0]
    left = 1.0 / (math.sqrt(2.0 * math.pi) * hh)
    inv2 = 1.0 / (2.0 * hh * hh)
    scalars = jnp.stack([left, inv2]).reshape(1, 2)

    We1a = We1[:D]
    We1b = We1[D:2 * D]
    w256 = We1[2 * D:]
    be1_r = be1.reshape(1, D)
    be2_r = be2.reshape(1, D)
    bn1_r = bn1.reshape(1, D)
    bn2_r = bn2.reshape(1, D)
    bl_r = b_lin.reshape(1, D)
    ba_r = ba.reshape(1, 1)
    Wn1a = Wn1[:D]
    Wn1b = Wn1[D:]

    A, B = _prep(h, We1a, We1b, be1_r)

    We2b = We2.astype(jnp.bfloat16)
    parts = []
    efs = []
    for lo, hi in bounds:
        r3 = row3d[lo // _EBLK:hi // _EBLK]
        c3 = col3d[lo // _EBLK:hi // _EBLK]
        pre = _gather_add(r3, c3, A, B, hi - lo, D)
        ef = _edge_mlp(scalars, pre, distances[lo:hi], edge_mask[lo:hi],
                       w256, We2b, be2_r, Wa, ba_r)
        efs.append((ef, r3))
    for ef, r3 in efs:
        p0, p1 = _scatter_add(ef, r3, N, D)
        parts.extend([p0, p1])
    hidden = _node_mlp(h, parts, Wn1a, Wn1b, bn1_r, Wn2, bn2_r, W_lin, bl_r)

    return (hidden, distances, edges, node_mask, edge_mask)
